# baseline (speedup 1.0000x reference)
import functools

import jax
import jax.numpy as jnp
from jax.experimental import pallas as pl
from jax.experimental.pallas import tpu as pltpu


def _scse_sample(x, w1s_ref, b1_ref, w2_ref, b2_ref, ws_ref, bs_ref):
    colsum = jnp.sum(x, axis=1, keepdims=True)

    pix_logit = jnp.dot(ws_ref[...], x,
                        precision=jax.lax.Precision.DEFAULT,
                        preferred_element_type=jnp.float32) + bs_ref[...]
    pix_gate = jax.nn.sigmoid(pix_logit)

    hidden = jnp.dot(w1s_ref[...], colsum,
                     preferred_element_type=jnp.float32) + b1_ref[...]
    hidden = jnp.maximum(hidden, 0.0)
    chan_logit = jnp.dot(w2_ref[...], hidden,
                         preferred_element_type=jnp.float32) + b2_ref[...]
    chan_gate = jax.nn.sigmoid(chan_logit)

    return x * (chan_gate + pix_gate)


def _scse_manual_body(x_hbm, w1s_ref, b1_ref, w2_ref, b2_ref, ws_ref, bs_ref,
                      o_hbm, x_buf, o_buf, in_sem, out_sem, *, grp):
    base = pl.program_id(0) * grp

    def dma_in(slot, k):
        return pltpu.make_async_copy(x_hbm.at[base + k], x_buf.at[slot],
                                     in_sem.at[slot])

    def dma_out(slot, k):
        return pltpu.make_async_copy(o_buf.at[slot], o_hbm.at[base + k],
                                     out_sem.at[slot])

    dma_in(0, 0).start()
    for k in range(grp):
        cur = k % 2
        if k + 1 < grp:
            dma_in(1 - cur, k + 1).start()
        dma_in(cur, k).wait()
        if k >= 2:
            dma_out(cur, k - 2).wait()
        o_buf[cur] = _scse_sample(x_buf[cur], w1s_ref, b1_ref, w2_ref,
                                  b2_ref, ws_ref, bs_ref)
        dma_out(cur, k).start()
    if grp >= 2:
        dma_out(grp % 2, grp - 2).wait()
    dma_out(1 - grp % 2, grp - 1).wait()


def kernel(x_nchw, w1, b1, w2, b2, ws, bs):
    N, C, H, W = x_nchw.shape
    HW = H * W
    Cr = w1.shape[0]
    x = x_nchw.reshape(N, C, HW)

    nsplit = 2 if N % 2 == 0 else 1
    grp = N // nsplit

    w1s = (w1.reshape(Cr, C) * (1.0 / float(HW))).astype(jnp.float32)
    b1c = b1.reshape(Cr, 1).astype(jnp.float32)
    w2m = w2.reshape(C, Cr).astype(jnp.float32)
    b2c = b2.reshape(C, 1).astype(jnp.float32)
    wsr = ws.reshape(1, C).astype(jnp.float32)
    bss = bs.reshape(1, 1).astype(jnp.float32)

    def whole(a):
        return pl.BlockSpec(a.shape, lambda s: (0,) * a.ndim)

    out = pl.pallas_call(
        functools.partial(_scse_manual_body, grp=grp),
        out_shape=jax.ShapeDtypeStruct((N, C, HW), x.dtype),
        grid=(nsplit,),
        in_specs=[pl.BlockSpec(memory_space=pl.ANY),
                  whole(w1s), whole(b1c), whole(w2m), whole(b2c),
                  whole(wsr), whole(bss)],
        out_specs=pl.BlockSpec(memory_space=pl.ANY),
        scratch_shapes=[pltpu.VMEM((2, C, HW), jnp.float32),
                        pltpu.VMEM((2, C, HW), jnp.float32),
                        pltpu.SemaphoreType.DMA((2,)),
                        pltpu.SemaphoreType.DMA((2,))],
        compiler_params=pltpu.CompilerParams(
            dimension_semantics=("parallel",),
            vmem_limit_bytes=96 << 20),
    )(x, w1s, b1c, w2m, b2c, wsr, bss)
    return out.reshape(N, C, H, W)


# speedup vs baseline: 1.0156x; 1.0156x over previous
import functools

import jax
import jax.numpy as jnp
from jax.experimental import pallas as pl
from jax.experimental.pallas import tpu as pltpu


def _scse_body(x_ref, w1s_ref, b1_ref, w2_ref, b2_ref, ws_ref, bs_ref, o_ref):
    x = x_ref[0]
    c, hw = x.shape

    colsum = jnp.sum(x, axis=1, keepdims=True)
    pix_logit = jnp.dot(ws_ref[...], x,
                        precision=jax.lax.Precision.DEFAULT,
                        preferred_element_type=jnp.float32) + bs_ref[...]
    pix_gate = jax.nn.sigmoid(pix_logit)

    hidden = jnp.dot(w1s_ref[...], colsum,
                     preferred_element_type=jnp.float32) + b1_ref[...]
    hidden = jnp.maximum(hidden, 0.0)
    chan_logit = jnp.dot(w2_ref[...], hidden,
                         preferred_element_type=jnp.float32) + b2_ref[...]
    chan_gate = jax.nn.sigmoid(chan_logit)

    o_ref[0] = x * (chan_gate + pix_gate)


def kernel(x_nchw, w1, b1, w2, b2, ws, bs):
    N, C, H, W = x_nchw.shape
    HW = H * W
    Cr = w1.shape[0]
    x = x_nchw.reshape(N, C, HW)

    w1s = (w1.reshape(Cr, C) * (1.0 / float(HW))).astype(jnp.float32)
    b1c = b1.reshape(Cr, 1).astype(jnp.float32)
    w2m = w2.reshape(C, Cr).astype(jnp.float32)
    b2c = b2.reshape(C, 1).astype(jnp.float32)
    wsr = ws.reshape(1, C).astype(jnp.float32)
    bss = bs.reshape(1, 1).astype(jnp.float32)

    sample_spec = pl.BlockSpec((1, C, HW), lambda n: (n, 0, 0))

    def whole(a):
        return pl.BlockSpec(a.shape, lambda n: (0,) * a.ndim)

    out = pl.pallas_call(
        _scse_body,
        out_shape=jax.ShapeDtypeStruct((N, C, HW), x.dtype),
        grid=(N,),
        in_specs=[sample_spec,
                  whole(w1s), whole(b1c), whole(w2m), whole(b2c),
                  whole(wsr), whole(bss)],
        out_specs=sample_spec,
        compiler_params=pltpu.CompilerParams(
            dimension_semantics=("parallel",),
            vmem_limit_bytes=96 << 20),
    )(x, w1s, b1c, w2m, b2c, wsr, bss)
    return out.reshape(N, C, H, W)
